# baseline (speedup 1.0000x reference)
import jax
import jax.numpy as jnp
from jax.experimental import pallas as pl
from jax.experimental.pallas import tpu as pltpu

_NUM_CHUNKS = 8


def _dma_copy(x_hbm, o_hbm, sem):
    rows = x_hbm.shape[0]
    chunk = rows // _NUM_CHUNKS
    for c in range(_NUM_CHUNKS):
        pltpu.make_async_copy(
            x_hbm.at[pl.ds(c * chunk, chunk)],
            o_hbm.at[pl.ds(c * chunk, chunk)],
            sem,
        ).start()
    for c in range(_NUM_CHUNKS):
        pltpu.make_async_copy(
            x_hbm.at[pl.ds(c * chunk, chunk)],
            o_hbm.at[pl.ds(c * chunk, chunk)],
            sem,
        ).wait()


def kernel(data, partitions):
    del partitions
    rows, cols = data.shape
    return pl.pallas_call(
        _dma_copy,
        in_specs=[pl.BlockSpec(memory_space=pl.ANY)],
        out_specs=pl.BlockSpec(memory_space=pl.ANY),
        scratch_shapes=[pltpu.SemaphoreType.DMA],
        out_shape=jax.ShapeDtypeStruct((rows, cols), data.dtype),
    )(data)


# speedup vs baseline: 49.1341x; 49.1341x over previous
import jax
import jax.numpy as jnp
from jax.experimental import pallas as pl
from jax.experimental.pallas import tpu as pltpu

_BLOCK_ROWS = 2048


def _copy_block(x_ref, o_ref):
    o_ref[...] = x_ref[...]


def kernel(data, partitions):
    del partitions
    rows, cols = data.shape
    return pl.pallas_call(
        _copy_block,
        grid=(rows // _BLOCK_ROWS,),
        in_specs=[pl.BlockSpec((_BLOCK_ROWS, cols), lambda i: (i, 0))],
        out_specs=pl.BlockSpec((_BLOCK_ROWS, cols), lambda i: (i, 0)),
        out_shape=jax.ShapeDtypeStruct((rows, cols), data.dtype),
        compiler_params=pltpu.CompilerParams(
            dimension_semantics=("parallel",),
            vmem_limit_bytes=100 * 1024 * 1024,
        ),
    )(data)
